# Initial kernel scaffold; baseline (speedup 1.0000x reference)
import jax
import jax.numpy as jnp
from jax.experimental import pallas as pl


def kernel(x_num, op_idx, edge_index, op_emb, W1, b1, W2, b2, gamma, beta):
    raise NotImplementedError("write your pallas kernel here")


# speedup vs baseline: 11.7928x; 11.7928x over previous
import functools

import jax
import jax.numpy as jnp
from jax import lax
from jax.experimental import pallas as pl
from jax.experimental.pallas import tpu as pltpu
from jax.experimental.pallas import tpu_sc as plsc

N = 50000
E = 1600000
N_OPS = 128
HIDDEN = 512

NC = 2
NS = 16
NW = NC * NS
N_PAD = 51200
RPS = N_PAD // NS

MB = 128
NMIC = 392
EPT = NMIC * MB
E_PAD = NW * EPT
EXT = E_PAD + MB
ZROWS = 800

BLK = 512
GRID = N_PAD // BLK


def _build_body(xn_ref, oi_ref, embp_ref, o_ref):
    idx = oi_ref[...]
    iot = lax.broadcasted_iota(jnp.int32, (BLK, N_OPS), 1)
    oh = (idx == iot).astype(jnp.float32)
    o_ref[...] = xn_ref[...] + jnp.dot(
        oh, embp_ref[...], preferred_element_type=jnp.float32)


def _build_x(xnum16, opidx2, embp):
    return pl.pallas_call(
        _build_body,
        grid=(GRID,),
        in_specs=[
            pl.BlockSpec((BLK, 16), lambda m: (m, 0)),
            pl.BlockSpec((BLK, 1), lambda m: (m, 0)),
            pl.BlockSpec((N_OPS, 16), lambda m: (0, 0)),
        ],
        out_specs=pl.BlockSpec((BLK, 16), lambda m: (m, 0)),
        out_shape=jax.ShapeDtypeStruct((N_PAD, 16), jnp.float32),
    )(xnum16, opidx2, embp)


def _edge_body(xpad, src1, dst1, agg_hbm,
               s_a, s_b, d_a, d_b, r_a, r_b, zbuf, aggs,
               gs_a, gs_b, is_a, is_b):
    c = lax.axis_index("c")
    s = lax.axis_index("s")
    wid = s * NC + c

    z16 = jnp.zeros((16,), jnp.float32)

    def _zrow(i, carry):
        zbuf[i, :] = z16
        return carry
    lax.fori_loop(0, ZROWS, _zrow, 0)
    sbase = pl.multiple_of(s * RPS, 8)
    for q in range(RPS // ZROWS):
        pltpu.sync_copy(zbuf, aggs.at[pl.ds(sbase + q * ZROWS, ZROWS)])
    plsc.subcore_barrier()

    e0 = pl.multiple_of(wid * EPT, 128)

    def idx_copy(m, sb, db, isem):
        off = pl.multiple_of(e0 + m * MB, 8)
        cp1 = pltpu.make_async_copy(src1.at[pl.ds(off, MB)], sb, isem)
        cp1.start()
        cp2 = pltpu.make_async_copy(dst1.at[pl.ds(off, MB)], db, isem)
        cp2.start()
        return cp1, cp2

    pltpu.sync_copy(src1.at[pl.ds(e0, MB)], s_a)
    pltpu.sync_copy(dst1.at[pl.ds(e0, MB)], d_a)
    pltpu.make_async_copy(xpad.at[s_a], r_a, gs_a).start()

    def _pair(p, carry):
        m0 = 2 * p
        cps, cpd = idx_copy(m0 + 1, s_b, d_b, is_b)
        pltpu.make_async_copy(xpad.at[s_a], r_a, gs_a).wait()
        pltpu.sync_copy(r_a, aggs.at[d_a], add=True)
        cps.wait()
        cpd.wait()
        pltpu.make_async_copy(xpad.at[s_b], r_b, gs_b).start()
        cps, cpd = idx_copy(m0 + 2, s_a, d_a, is_a)
        pltpu.make_async_copy(xpad.at[s_b], r_b, gs_b).wait()
        pltpu.sync_copy(r_b, aggs.at[d_b], add=True)
        cps.wait()
        cpd.wait()
        pltpu.make_async_copy(xpad.at[s_a], r_a, gs_a).start()
        return carry
    lax.fori_loop(0, NMIC // 2, _pair, 0)
    pltpu.make_async_copy(xpad.at[s_a], r_a, gs_a).wait()

    plsc.subcore_barrier()
    pltpu.sync_copy(aggs.at[pl.ds(sbase, RPS)],
                    agg_hbm.at[c, pl.ds(sbase, RPS)])


def _edge_agg(x_pad, src1, dst1):
    mesh = plsc.VectorSubcoreMesh(core_axis_name="c", subcore_axis_name="s")
    fn = functools.partial(
        pl.kernel,
        out_type=jax.ShapeDtypeStruct((NC, N_PAD, 16), jnp.float32),
        mesh=mesh,
        compiler_params=pltpu.CompilerParams(use_tc_tiling_on_sc=False),
        scratch_types=[
            pltpu.VMEM((MB,), jnp.int32),
            pltpu.VMEM((MB,), jnp.int32),
            pltpu.VMEM((MB,), jnp.int32),
            pltpu.VMEM((MB,), jnp.int32),
            pltpu.VMEM((MB, 16), jnp.float32),
            pltpu.VMEM((MB, 16), jnp.float32),
            pltpu.VMEM((ZROWS, 16), jnp.float32),
            pltpu.VMEM_SHARED((N_PAD, 16), jnp.float32),
            pltpu.SemaphoreType.DMA,
            pltpu.SemaphoreType.DMA,
            pltpu.SemaphoreType.DMA,
            pltpu.SemaphoreType.DMA,
        ],
    )(_edge_body)
    return fn(x_pad, src1, dst1)


def _mlp_body(x_ref, a0_ref, a1_ref, w1_ref, b1_ref, w2_ref, b2_ref,
              g_ref, be_ref, o_ref):
    m = pl.program_id(0)
    h0 = x_ref[...] + a0_ref[...] + a1_ref[...]
    h1 = jnp.maximum(
        jnp.dot(h0, w1_ref[...], preferred_element_type=jnp.float32)
        + b1_ref[...], 0.0)
    h2 = jnp.dot(h1, w2_ref[...], preferred_element_type=jnp.float32) \
        + b2_ref[...]
    mu = jnp.mean(h2, axis=-1, keepdims=True)
    d = h2 - mu
    var = jnp.mean(d * d, axis=-1, keepdims=True)
    hn = d * lax.rsqrt(var + 1e-5) * g_ref[...] + be_ref[...]
    rows = m * BLK + lax.broadcasted_iota(jnp.int32, (BLK, 1), 0)
    hn = jnp.where(rows < N, hn, 0.0)

    @pl.when(m == 0)
    def _():
        o_ref[...] = jnp.zeros_like(o_ref)
    o_ref[...] += jnp.sum(hn, axis=0, keepdims=True)

    @pl.when(m == GRID - 1)
    def _():
        o_ref[...] *= (1.0 / N)


def _mlp_mean(x_pad, agg0, agg1, w1p, b1, w2, b2, gamma, beta):
    blk16 = pl.BlockSpec((BLK, 16), lambda m: (m, 0))
    row512 = pl.BlockSpec((1, HIDDEN), lambda m: (0, 0))
    return pl.pallas_call(
        _mlp_body,
        grid=(GRID,),
        in_specs=[
            blk16, blk16, blk16,
            pl.BlockSpec((16, HIDDEN), lambda m: (0, 0)),
            row512,
            pl.BlockSpec((HIDDEN, HIDDEN), lambda m: (0, 0)),
            row512, row512, row512,
        ],
        out_specs=pl.BlockSpec((1, HIDDEN), lambda m: (0, 0)),
        out_shape=jax.ShapeDtypeStruct((1, HIDDEN), jnp.float32),
    )(x_pad, agg0, agg1, w1p, b1, w2, b2, gamma, beta)


@jax.jit
def kernel(x_num, op_idx, edge_index, op_emb, W1, b1, W2, b2, gamma, beta):
    op_idx = op_idx.astype(jnp.int32)
    edge_index = edge_index.astype(jnp.int32)

    xnum16 = jnp.zeros((N_PAD, 16), jnp.float32).at[:N, 0:2].set(x_num)
    opidx2 = jnp.full((N_PAD, 1), N_OPS, jnp.int32).at[:N, 0].set(op_idx)
    embp = jnp.zeros((N_OPS, 16), jnp.float32).at[:, 2:10].set(op_emb)
    w1p = jnp.zeros((16, HIDDEN), jnp.float32).at[0:10, :].set(W1)

    src1 = jnp.full((EXT,), N, jnp.int32).at[:E].set(edge_index[0])
    dst1 = jnp.full((EXT,), N, jnp.int32).at[:E].set(edge_index[1])

    x_pad = _build_x(xnum16, opidx2, embp)
    agg = _edge_agg(x_pad, src1, dst1)
    out = _mlp_mean(x_pad, agg[0], agg[1],
                    w1p, b1.reshape(1, HIDDEN), W2,
                    b2.reshape(1, HIDDEN), gamma.reshape(1, HIDDEN),
                    beta.reshape(1, HIDDEN))
    return out[0]


# speedup vs baseline: 13.2254x; 1.1215x over previous
import functools

import jax
import jax.numpy as jnp
from jax import lax
from jax.experimental import pallas as pl
from jax.experimental.pallas import tpu as pltpu
from jax.experimental.pallas import tpu_sc as plsc

N = 50000
E = 1600000
N_OPS = 128
HIDDEN = 512

NC = 2
NS = 16
NW = NC * NS

BLK = 512
GRID = 98
N_PAD = GRID * BLK
RPS = N_PAD // NS
ZROWS = 784

MB = 128
EPT = E // NW
NFULL = EPT // MB
TAIL = EPT - NFULL * MB
NPAIR = NFULL // 2


def _build_body(xn_ref, oi_ref, embp_ref, o_ref):
    idx = oi_ref[...]
    iot = lax.broadcasted_iota(jnp.int32, (BLK, N_OPS), 1)
    oh = (idx == iot).astype(jnp.float32)
    xn = jnp.concatenate(
        [xn_ref[...], jnp.zeros((BLK, 14), jnp.float32)], axis=1)
    o_ref[...] = xn + jnp.dot(
        oh, embp_ref[...], preferred_element_type=jnp.float32)


def _build_x(x_num, opidx2, embp):
    return pl.pallas_call(
        _build_body,
        grid=(GRID,),
        in_specs=[
            pl.BlockSpec((BLK, 2), lambda m: (m, 0)),
            pl.BlockSpec((BLK, 1), lambda m: (m, 0)),
            pl.BlockSpec((N_OPS, 16), lambda m: (0, 0)),
        ],
        out_specs=pl.BlockSpec((BLK, 16), lambda m: (m, 0)),
        out_shape=jax.ShapeDtypeStruct((N_PAD, 16), jnp.float32),
    )(x_num, opidx2, embp)


def _edge_body(xpad, src1, dst1, agg_hbm,
               s_a, s_b, d_a, d_b, r_a, r_b, s_t, d_t, r_t, zbuf, aggs,
               gs_a, gs_b, is_a, is_b):
    c = lax.axis_index("c")
    s = lax.axis_index("s")
    wid = s * NC + c

    z16 = jnp.zeros((16,), jnp.float32)

    def _zrow(i, carry):
        zbuf[i, :] = z16
        return carry
    lax.fori_loop(0, ZROWS, _zrow, 0)
    sbase = pl.multiple_of(s * RPS, 8)
    for q in range(RPS // ZROWS):
        pltpu.sync_copy(zbuf, aggs.at[pl.ds(sbase + q * ZROWS, ZROWS)])
    plsc.subcore_barrier()

    e0 = pl.multiple_of(wid * EPT, 8)

    def idx_copy(m, sb, db, isem):
        off = pl.multiple_of(e0 + m * MB, 8)
        cp1 = pltpu.make_async_copy(src1.at[pl.ds(off, MB)], sb, isem)
        cp1.start()
        cp2 = pltpu.make_async_copy(dst1.at[pl.ds(off, MB)], db, isem)
        cp2.start()
        return cp1, cp2

    pltpu.sync_copy(src1.at[pl.ds(e0, MB)], s_a)
    pltpu.sync_copy(dst1.at[pl.ds(e0, MB)], d_a)
    pltpu.make_async_copy(xpad.at[s_a], r_a, gs_a).start()

    def _pair(p, carry):
        m0 = 2 * p
        cps, cpd = idx_copy(m0 + 1, s_b, d_b, is_b)
        pltpu.make_async_copy(xpad.at[s_a], r_a, gs_a).wait()
        pltpu.sync_copy(r_a, aggs.at[d_a], add=True)
        cps.wait()
        cpd.wait()
        pltpu.make_async_copy(xpad.at[s_b], r_b, gs_b).start()
        cps, cpd = idx_copy(m0 + 2, s_a, d_a, is_a)
        pltpu.make_async_copy(xpad.at[s_b], r_b, gs_b).wait()
        pltpu.sync_copy(r_b, aggs.at[d_b], add=True)
        cps.wait()
        cpd.wait()
        pltpu.make_async_copy(xpad.at[s_a], r_a, gs_a).start()
        return carry
    lax.fori_loop(0, NPAIR - 1, _pair, 0)

    cps, cpd = idx_copy(NFULL - 1, s_b, d_b, is_b)
    pltpu.make_async_copy(xpad.at[s_a], r_a, gs_a).wait()
    pltpu.sync_copy(r_a, aggs.at[d_a], add=True)
    cps.wait()
    cpd.wait()
    pltpu.make_async_copy(xpad.at[s_b], r_b, gs_b).start()
    pltpu.make_async_copy(xpad.at[s_b], r_b, gs_b).wait()
    pltpu.sync_copy(r_b, aggs.at[d_b], add=True)

    off_t = pl.multiple_of(e0 + NFULL * MB, 8)
    pltpu.sync_copy(src1.at[pl.ds(off_t, TAIL)], s_t)
    pltpu.sync_copy(dst1.at[pl.ds(off_t, TAIL)], d_t)
    cp_t = pltpu.make_async_copy(xpad.at[s_t], r_t, gs_a)
    cp_t.start()
    cp_t.wait()
    pltpu.sync_copy(r_t, aggs.at[d_t], add=True)

    plsc.subcore_barrier()
    pltpu.sync_copy(aggs.at[pl.ds(sbase, RPS)],
                    agg_hbm.at[c, pl.ds(sbase, RPS)])


def _edge_agg(x_pad, src1, dst1):
    mesh = plsc.VectorSubcoreMesh(core_axis_name="c", subcore_axis_name="s")
    fn = functools.partial(
        pl.kernel,
        out_type=jax.ShapeDtypeStruct((NC, N_PAD, 16), jnp.float32),
        mesh=mesh,
        compiler_params=pltpu.CompilerParams(use_tc_tiling_on_sc=False),
        scratch_types=[
            pltpu.VMEM((MB,), jnp.int32),
            pltpu.VMEM((MB,), jnp.int32),
            pltpu.VMEM((MB,), jnp.int32),
            pltpu.VMEM((MB,), jnp.int32),
            pltpu.VMEM((MB, 16), jnp.float32),
            pltpu.VMEM((MB, 16), jnp.float32),
            pltpu.VMEM((TAIL,), jnp.int32),
            pltpu.VMEM((TAIL,), jnp.int32),
            pltpu.VMEM((TAIL, 16), jnp.float32),
            pltpu.VMEM((ZROWS, 16), jnp.float32),
            pltpu.VMEM_SHARED((N_PAD, 16), jnp.float32),
            pltpu.SemaphoreType.DMA,
            pltpu.SemaphoreType.DMA,
            pltpu.SemaphoreType.DMA,
            pltpu.SemaphoreType.DMA,
        ],
    )(_edge_body)
    return fn(x_pad, src1, dst1)


def _mlp_body(x_ref, a_ref, w1_ref, b1_ref, w2_ref, b2_ref,
              g_ref, be_ref, o_ref):
    m = pl.program_id(0)
    h0 = x_ref[...] + a_ref[0] + a_ref[1]
    h1 = jnp.maximum(
        jnp.dot(h0, w1_ref[...], preferred_element_type=jnp.float32)
        + b1_ref[...], 0.0)
    h2 = jnp.dot(h1, w2_ref[...], preferred_element_type=jnp.float32) \
        + b2_ref[...]
    mu = jnp.mean(h2, axis=-1, keepdims=True)
    d = h2 - mu
    var = jnp.mean(d * d, axis=-1, keepdims=True)
    hn = d * lax.rsqrt(var + 1e-5) * g_ref[...] + be_ref[...]
    rows = m * BLK + lax.broadcasted_iota(jnp.int32, (BLK, 1), 0)
    hn = jnp.where(rows < N, hn, 0.0)

    @pl.when(m == 0)
    def _():
        o_ref[...] = jnp.zeros_like(o_ref)
    o_ref[...] += jnp.sum(hn, axis=0, keepdims=True)

    @pl.when(m == GRID - 1)
    def _():
        o_ref[...] *= (1.0 / N)


def _mlp_mean(x_pad, agg, w1p, b1, w2, b2, gamma, beta):
    row512 = pl.BlockSpec((1, HIDDEN), lambda m: (0, 0))
    return pl.pallas_call(
        _mlp_body,
        grid=(GRID,),
        in_specs=[
            pl.BlockSpec((BLK, 16), lambda m: (m, 0)),
            pl.BlockSpec((NC, BLK, 16), lambda m: (0, m, 0)),
            pl.BlockSpec((16, HIDDEN), lambda m: (0, 0)),
            row512,
            pl.BlockSpec((HIDDEN, HIDDEN), lambda m: (0, 0)),
            row512, row512, row512,
        ],
        out_specs=pl.BlockSpec((1, HIDDEN), lambda m: (0, 0)),
        out_shape=jax.ShapeDtypeStruct((1, HIDDEN), jnp.float32),
    )(x_pad, agg, w1p, b1, w2, b2, gamma, beta)


@jax.jit
def kernel(x_num, op_idx, edge_index, op_emb, W1, b1, W2, b2, gamma, beta):
    op_idx = op_idx.astype(jnp.int32)
    edge_index = edge_index.astype(jnp.int32)

    opidx2 = op_idx.reshape(N, 1)
    embp = jnp.zeros((N_OPS, 16), jnp.float32).at[:, 2:10].set(op_emb)
    w1p = jnp.zeros((16, HIDDEN), jnp.float32).at[0:10, :].set(W1)

    x_pad = _build_x(x_num, opidx2, embp)
    agg = _edge_agg(x_pad, edge_index[0], edge_index[1])
    out = _mlp_mean(x_pad, agg,
                    w1p, b1.reshape(1, HIDDEN), W2,
                    b2.reshape(1, HIDDEN), gamma.reshape(1, HIDDEN),
                    beta.reshape(1, HIDDEN))
    return out[0]


# speedup vs baseline: 14.2760x; 1.0794x over previous
import functools

import jax
import jax.numpy as jnp
from jax import lax
from jax.experimental import pallas as pl
from jax.experimental.pallas import tpu as pltpu
from jax.experimental.pallas import tpu_sc as plsc

N = 50000
E = 1600000
N_OPS = 128
HIDDEN = 512

NC = 2
NS = 16
NW = NC * NS

BLK = 512
GRID = 98
N_PAD = GRID * BLK
RPS = N_PAD // NS
ZROWS = 784

MB = 128
EPT = E // NW
NFULL = EPT // MB
TAIL = EPT - NFULL * MB
NPAIR = NFULL // 2


def _build_body(xn_ref, oi_ref, embp_ref, o_ref):
    idx = oi_ref[...]
    iot = lax.broadcasted_iota(jnp.int32, (BLK, N_OPS), 1)
    oh = (idx == iot).astype(jnp.float32)
    xn = jnp.concatenate(
        [xn_ref[...], jnp.zeros((BLK, 14), jnp.float32)], axis=1)
    o_ref[...] = xn + jnp.dot(
        oh, embp_ref[...], preferred_element_type=jnp.float32)


def _build_x(x_num, opidx2, embp):
    return pl.pallas_call(
        _build_body,
        grid=(GRID,),
        in_specs=[
            pl.BlockSpec((BLK, 2), lambda m: (m, 0)),
            pl.BlockSpec((BLK, 1), lambda m: (m, 0)),
            pl.BlockSpec((N_OPS, 16), lambda m: (0, 0)),
        ],
        out_specs=pl.BlockSpec((BLK, 16), lambda m: (m, 0)),
        out_shape=jax.ShapeDtypeStruct((N_PAD, 16), jnp.float32),
    )(x_num, opidx2, embp)


def _edge_body(xpad, src1, dst1, agg_hbm,
               s_a, s_b, d_a, d_b, r_a, r_b, s_t, d_t, r_t, zbuf, aggs,
               gs_a, gs_b, is_a, is_b, ss_a, ss_b):
    c = lax.axis_index("c")
    s = lax.axis_index("s")
    wid = s * NC + c

    z16 = jnp.zeros((16,), jnp.float32)

    def _zrow(i, carry):
        zbuf[i, :] = z16
        return carry
    lax.fori_loop(0, ZROWS, _zrow, 0)
    sbase = pl.multiple_of(s * RPS, 8)
    for q in range(RPS // ZROWS):
        pltpu.sync_copy(zbuf, aggs.at[pl.ds(sbase + q * ZROWS, ZROWS)])
    plsc.subcore_barrier()

    e0 = pl.multiple_of(wid * EPT, 8)

    def idx_copy(m, sb, db, isem):
        off = pl.multiple_of(e0 + m * MB, 8)
        cp1 = pltpu.make_async_copy(src1.at[pl.ds(off, MB)], sb, isem)
        cp1.start()
        cp2 = pltpu.make_async_copy(dst1.at[pl.ds(off, MB)], db, isem)
        cp2.start()
        return cp1, cp2

    pltpu.sync_copy(src1.at[pl.ds(e0, MB)], s_a)
    pltpu.sync_copy(dst1.at[pl.ds(e0, MB)], d_a)
    pltpu.make_async_copy(xpad.at[s_a], r_a, gs_a).start()

    def _pair(p, carry):
        m0 = 2 * p
        cps, cpd = idx_copy(m0 + 1, s_b, d_b, is_b)
        pltpu.make_async_copy(xpad.at[s_a], r_a, gs_a).wait()
        sc_a = pltpu.async_copy(r_a, aggs.at[d_a], ss_a, add=True)
        cps.wait()
        cpd.wait()
        pltpu.make_async_copy(xpad.at[s_b], r_b, gs_b).start()
        sc_a.wait()
        cps, cpd = idx_copy(m0 + 2, s_a, d_a, is_a)
        pltpu.make_async_copy(xpad.at[s_b], r_b, gs_b).wait()
        sc_b = pltpu.async_copy(r_b, aggs.at[d_b], ss_b, add=True)
        cps.wait()
        cpd.wait()
        pltpu.make_async_copy(xpad.at[s_a], r_a, gs_a).start()
        sc_b.wait()
        return carry
    lax.fori_loop(0, NPAIR - 1, _pair, 0)

    cps, cpd = idx_copy(NFULL - 1, s_b, d_b, is_b)
    pltpu.make_async_copy(xpad.at[s_a], r_a, gs_a).wait()
    pltpu.sync_copy(r_a, aggs.at[d_a], add=True)
    cps.wait()
    cpd.wait()
    pltpu.make_async_copy(xpad.at[s_b], r_b, gs_b).start()
    pltpu.make_async_copy(xpad.at[s_b], r_b, gs_b).wait()
    pltpu.sync_copy(r_b, aggs.at[d_b], add=True)

    off_t = pl.multiple_of(e0 + NFULL * MB, 8)
    pltpu.sync_copy(src1.at[pl.ds(off_t, TAIL)], s_t)
    pltpu.sync_copy(dst1.at[pl.ds(off_t, TAIL)], d_t)
    cp_t = pltpu.make_async_copy(xpad.at[s_t], r_t, gs_a)
    cp_t.start()
    cp_t.wait()
    pltpu.sync_copy(r_t, aggs.at[d_t], add=True)

    plsc.subcore_barrier()
    pltpu.sync_copy(aggs.at[pl.ds(sbase, RPS)],
                    agg_hbm.at[c, pl.ds(sbase, RPS)])


def _edge_agg(x_pad, src1, dst1):
    mesh = plsc.VectorSubcoreMesh(core_axis_name="c", subcore_axis_name="s")
    fn = functools.partial(
        pl.kernel,
        out_type=jax.ShapeDtypeStruct((NC, N_PAD, 16), jnp.float32),
        mesh=mesh,
        compiler_params=pltpu.CompilerParams(use_tc_tiling_on_sc=False),
        scratch_types=[
            pltpu.VMEM((MB,), jnp.int32),
            pltpu.VMEM((MB,), jnp.int32),
            pltpu.VMEM((MB,), jnp.int32),
            pltpu.VMEM((MB,), jnp.int32),
            pltpu.VMEM((MB, 16), jnp.float32),
            pltpu.VMEM((MB, 16), jnp.float32),
            pltpu.VMEM((TAIL,), jnp.int32),
            pltpu.VMEM((TAIL,), jnp.int32),
            pltpu.VMEM((TAIL, 16), jnp.float32),
            pltpu.VMEM((ZROWS, 16), jnp.float32),
            pltpu.VMEM_SHARED((N_PAD, 16), jnp.float32),
            pltpu.SemaphoreType.DMA,
            pltpu.SemaphoreType.DMA,
            pltpu.SemaphoreType.DMA,
            pltpu.SemaphoreType.DMA,
            pltpu.SemaphoreType.DMA,
            pltpu.SemaphoreType.DMA,
        ],
    )(_edge_body)
    return fn(x_pad, src1, dst1)


def _mlp_body(x_ref, a_ref, w1_ref, b1_ref, w2_ref, b2_ref,
              g_ref, be_ref, o_ref):
    m = pl.program_id(0)
    h0 = x_ref[...] + a_ref[0] + a_ref[1]
    h1 = jnp.maximum(
        jnp.dot(h0, w1_ref[...], preferred_element_type=jnp.float32)
        + b1_ref[...], 0.0)
    h2 = jnp.dot(h1.astype(jnp.bfloat16), w2_ref[...],
                 preferred_element_type=jnp.float32) + b2_ref[...]
    mu = jnp.mean(h2, axis=-1, keepdims=True)
    d = h2 - mu
    var = jnp.mean(d * d, axis=-1, keepdims=True)
    hn = d * lax.rsqrt(var + 1e-5) * g_ref[...] + be_ref[...]
    rows = m * BLK + lax.broadcasted_iota(jnp.int32, (BLK, 1), 0)
    hn = jnp.where(rows < N, hn, 0.0)

    @pl.when(m == 0)
    def _():
        o_ref[...] = jnp.zeros_like(o_ref)
    o_ref[...] += jnp.sum(hn, axis=0, keepdims=True)

    @pl.when(m == GRID - 1)
    def _():
        o_ref[...] *= (1.0 / N)


def _mlp_mean(x_pad, agg, w1p, b1, w2, b2, gamma, beta):
    row512 = pl.BlockSpec((1, HIDDEN), lambda m: (0, 0))
    return pl.pallas_call(
        _mlp_body,
        grid=(GRID,),
        in_specs=[
            pl.BlockSpec((BLK, 16), lambda m: (m, 0)),
            pl.BlockSpec((NC, BLK, 16), lambda m: (0, m, 0)),
            pl.BlockSpec((16, HIDDEN), lambda m: (0, 0)),
            row512,
            pl.BlockSpec((HIDDEN, HIDDEN), lambda m: (0, 0)),
            row512, row512, row512,
        ],
        out_specs=pl.BlockSpec((1, HIDDEN), lambda m: (0, 0)),
        out_shape=jax.ShapeDtypeStruct((1, HIDDEN), jnp.float32),
    )(x_pad, agg, w1p, b1, w2, b2, gamma, beta)


@jax.jit
def kernel(x_num, op_idx, edge_index, op_emb, W1, b1, W2, b2, gamma, beta):
    op_idx = op_idx.astype(jnp.int32)
    edge_index = edge_index.astype(jnp.int32)

    opidx2 = op_idx.reshape(N, 1)
    embp = jnp.zeros((N_OPS, 16), jnp.float32).at[:, 2:10].set(op_emb)
    w1p = jnp.zeros((16, HIDDEN), jnp.float32).at[0:10, :].set(W1)

    x_pad = _build_x(x_num, opidx2, embp)
    agg = _edge_agg(x_pad, edge_index[0], edge_index[1])
    out = _mlp_mean(x_pad, agg,
                    w1p, b1.reshape(1, HIDDEN), W2.astype(jnp.bfloat16),
                    b2.reshape(1, HIDDEN), gamma.reshape(1, HIDDEN),
                    beta.reshape(1, HIDDEN))
    return out[0]


# speedup vs baseline: 16.3282x; 1.1437x over previous
import functools

import jax
import jax.numpy as jnp
from jax import lax
from jax.experimental import pallas as pl
from jax.experimental.pallas import tpu as pltpu
from jax.experimental.pallas import tpu_sc as plsc

N = 50000
E = 1600000
N_OPS = 128
HIDDEN = 512

NC = 2
NS = 16
NW = NC * NS

BLK = 512
GRID = 98
N_PAD = GRID * BLK
RPS = N_PAD // NS
ZROWS = 784

MB = 128
EPT = E // NW
NFULL = EPT // MB
TAIL = EPT - NFULL * MB
NPAIR = NFULL // 2
RPB = N_PAD // NW


def _build_body(xnum, opidx, emb, xs_out, xnb, oib, embv, xb):
    c = lax.axis_index("c")
    s = lax.axis_index("s")
    wid = s * NC + c
    r0 = pl.multiple_of(wid * RPB, 8)
    pltpu.sync_copy(xnum.at[pl.ds(r0, RPB)], xnb)
    pltpu.sync_copy(opidx.at[pl.ds(r0, RPB)], oib)
    pltpu.sync_copy(emb, embv)

    iota16 = lax.broadcasted_iota(jnp.int32, (16,), 0)
    z16 = jnp.zeros((16,), jnp.float32)

    def _bld(g, carry):
        rows16 = g * 16 + iota16
        idx16 = oib[pl.ds(g * 16, 16)]
        for col in range(2):
            cc = jnp.full((16,), col, jnp.int32)
            v = plsc.load_gather(xnb, [rows16, cc])
            plsc.store_scatter(xb, [rows16, cc], v)
        for ec in range(8):
            v = plsc.load_gather(embv, [idx16, jnp.full((16,), ec, jnp.int32)])
            plsc.store_scatter(
                xb, [rows16, jnp.full((16,), 2 + ec, jnp.int32)], v)
        for zc in range(10, 16):
            plsc.store_scatter(
                xb, [rows16, jnp.full((16,), zc, jnp.int32)], z16)
        return carry
    lax.fori_loop(0, RPB // 16, _bld, 0)
    pltpu.sync_copy(xb, xs_out.at[pl.ds(r0, RPB)])


def _build_x(xnum_p, opidx_p, emb):
    mesh = plsc.VectorSubcoreMesh(core_axis_name="c", subcore_axis_name="s")
    fn = functools.partial(
        pl.kernel,
        out_type=jax.ShapeDtypeStruct((N_PAD, 16), jnp.float32),
        mesh=mesh,
        compiler_params=pltpu.CompilerParams(
            use_tc_tiling_on_sc=False, needs_layout_passes=False),
        scratch_types=[
            pltpu.VMEM((RPB, 2), jnp.float32),
            pltpu.VMEM((RPB,), jnp.int32),
            pltpu.VMEM((N_OPS, 8), jnp.float32),
            pltpu.VMEM((RPB, 16), jnp.float32),
        ],
    )(_build_body)
    return fn(xnum_p, opidx_p, emb)


def _edge_body(xpad, src1, dst1, agg_hbm,
               s_a, s_b, d_a, d_b, r_a, r_b, s_t, d_t, r_t, zbuf, aggs,
               gs_a, gs_b, is_a, is_b, ss_a, ss_b):
    c = lax.axis_index("c")
    s = lax.axis_index("s")
    wid = s * NC + c

    sbase = pl.multiple_of(s * RPS, 8)

    @pl.when(c == 0)
    def _():
        pltpu.sync_copy(xpad.at[pl.ds(sbase, RPS)],
                        aggs.at[pl.ds(sbase, RPS)])

    @pl.when(c != 0)
    def _():
        z16 = jnp.zeros((16,), jnp.float32)

        def _zrow(i, carry):
            zbuf[i, :] = z16
            return carry
        lax.fori_loop(0, ZROWS, _zrow, 0)
        for q in range(RPS // ZROWS):
            pltpu.sync_copy(zbuf, aggs.at[pl.ds(sbase + q * ZROWS, ZROWS)])
    plsc.subcore_barrier()

    e0 = pl.multiple_of(wid * EPT, 8)

    def idx_copy(m, sb, db, isem):
        off = pl.multiple_of(e0 + m * MB, 8)
        cp1 = pltpu.make_async_copy(src1.at[pl.ds(off, MB)], sb, isem)
        cp1.start()
        cp2 = pltpu.make_async_copy(dst1.at[pl.ds(off, MB)], db, isem)
        cp2.start()
        return cp1, cp2

    pltpu.sync_copy(src1.at[pl.ds(e0, MB)], s_a)
    pltpu.sync_copy(dst1.at[pl.ds(e0, MB)], d_a)
    pltpu.make_async_copy(xpad.at[s_a], r_a, gs_a).start()

    def _pair(p, carry):
        m0 = 2 * p
        cps, cpd = idx_copy(m0 + 1, s_b, d_b, is_b)
        pltpu.make_async_copy(xpad.at[s_a], r_a, gs_a).wait()
        sc_a = pltpu.async_copy(r_a, aggs.at[d_a], ss_a, add=True)
        cps.wait()
        cpd.wait()
        pltpu.make_async_copy(xpad.at[s_b], r_b, gs_b).start()
        sc_a.wait()
        cps, cpd = idx_copy(m0 + 2, s_a, d_a, is_a)
        pltpu.make_async_copy(xpad.at[s_b], r_b, gs_b).wait()
        sc_b = pltpu.async_copy(r_b, aggs.at[d_b], ss_b, add=True)
        cps.wait()
        cpd.wait()
        pltpu.make_async_copy(xpad.at[s_a], r_a, gs_a).start()
        sc_b.wait()
        return carry
    lax.fori_loop(0, NPAIR - 1, _pair, 0)

    cps, cpd = idx_copy(NFULL - 1, s_b, d_b, is_b)
    pltpu.make_async_copy(xpad.at[s_a], r_a, gs_a).wait()
    pltpu.sync_copy(r_a, aggs.at[d_a], add=True)
    cps.wait()
    cpd.wait()
    pltpu.make_async_copy(xpad.at[s_b], r_b, gs_b).start()
    pltpu.make_async_copy(xpad.at[s_b], r_b, gs_b).wait()
    pltpu.sync_copy(r_b, aggs.at[d_b], add=True)

    off_t = pl.multiple_of(e0 + NFULL * MB, 8)
    pltpu.sync_copy(src1.at[pl.ds(off_t, TAIL)], s_t)
    pltpu.sync_copy(dst1.at[pl.ds(off_t, TAIL)], d_t)
    cp_t = pltpu.make_async_copy(xpad.at[s_t], r_t, gs_a)
    cp_t.start()
    cp_t.wait()
    pltpu.sync_copy(r_t, aggs.at[d_t], add=True)

    plsc.subcore_barrier()
    pltpu.sync_copy(aggs.at[pl.ds(sbase, RPS)],
                    agg_hbm.at[c, pl.ds(sbase, RPS)])


def _edge_agg(x_pad, src1, dst1):
    mesh = plsc.VectorSubcoreMesh(core_axis_name="c", subcore_axis_name="s")
    fn = functools.partial(
        pl.kernel,
        out_type=jax.ShapeDtypeStruct((NC, N_PAD, 16), jnp.float32),
        mesh=mesh,
        compiler_params=pltpu.CompilerParams(use_tc_tiling_on_sc=False),
        scratch_types=[
            pltpu.VMEM((MB,), jnp.int32),
            pltpu.VMEM((MB,), jnp.int32),
            pltpu.VMEM((MB,), jnp.int32),
            pltpu.VMEM((MB,), jnp.int32),
            pltpu.VMEM((MB, 16), jnp.float32),
            pltpu.VMEM((MB, 16), jnp.float32),
            pltpu.VMEM((TAIL,), jnp.int32),
            pltpu.VMEM((TAIL,), jnp.int32),
            pltpu.VMEM((TAIL, 16), jnp.float32),
            pltpu.VMEM((ZROWS, 16), jnp.float32),
            pltpu.VMEM_SHARED((N_PAD, 16), jnp.float32),
            pltpu.SemaphoreType.DMA,
            pltpu.SemaphoreType.DMA,
            pltpu.SemaphoreType.DMA,
            pltpu.SemaphoreType.DMA,
            pltpu.SemaphoreType.DMA,
            pltpu.SemaphoreType.DMA,
        ],
    )(_edge_body)
    return fn(x_pad, src1, dst1)


def _mlp_body(a_ref, w1_ref, b1_ref, w2_ref, b2_ref,
              g_ref, be_ref, o_ref):
    m = pl.program_id(0)
    h0 = a_ref[0] + a_ref[1]
    h1 = jnp.maximum(
        jnp.dot(h0, w1_ref[...], preferred_element_type=jnp.float32)
        + b1_ref[...], 0.0)
    h2 = jnp.dot(h1.astype(jnp.bfloat16), w2_ref[...],
                 preferred_element_type=jnp.float32) + b2_ref[...]
    mu = jnp.mean(h2, axis=-1, keepdims=True)
    d = h2 - mu
    var = jnp.mean(d * d, axis=-1, keepdims=True)
    hn = d * lax.rsqrt(var + 1e-5) * g_ref[...] + be_ref[...]
    rows = m * BLK + lax.broadcasted_iota(jnp.int32, (BLK, 1), 0)
    hn = jnp.where(rows < N, hn, 0.0)

    @pl.when(m == 0)
    def _():
        o_ref[...] = jnp.zeros_like(o_ref)
    o_ref[...] += jnp.sum(hn, axis=0, keepdims=True)

    @pl.when(m == GRID - 1)
    def _():
        o_ref[...] *= (1.0 / N)


def _mlp_mean(agg, w1p, b1, w2, b2, gamma, beta):
    row512 = pl.BlockSpec((1, HIDDEN), lambda m: (0, 0))
    return pl.pallas_call(
        _mlp_body,
        grid=(GRID,),
        in_specs=[
            pl.BlockSpec((NC, BLK, 16), lambda m: (0, m, 0)),
            pl.BlockSpec((16, HIDDEN), lambda m: (0, 0)),
            row512,
            pl.BlockSpec((HIDDEN, HIDDEN), lambda m: (0, 0)),
            row512, row512, row512,
        ],
        out_specs=pl.BlockSpec((1, HIDDEN), lambda m: (0, 0)),
        out_shape=jax.ShapeDtypeStruct((1, HIDDEN), jnp.float32),
    )(agg, w1p, b1, w2, b2, gamma, beta)


@jax.jit
def kernel(x_num, op_idx, edge_index, op_emb, W1, b1, W2, b2, gamma, beta):
    op_idx = op_idx.astype(jnp.int32)
    edge_index = edge_index.astype(jnp.int32)

    xnum_p = jnp.zeros((N_PAD, 2), jnp.float32).at[:N].set(x_num)
    opidx_p = jnp.zeros((N_PAD,), jnp.int32).at[:N].set(op_idx)
    w1p = jnp.zeros((16, HIDDEN), jnp.float32).at[0:10, :].set(W1)

    x_pad = _build_x(xnum_p, opidx_p, op_emb)
    agg = _edge_agg(x_pad, edge_index[0], edge_index[1])
    out = _mlp_mean(agg,
                    w1p, b1.reshape(1, HIDDEN), W2.astype(jnp.bfloat16),
                    b2.reshape(1, HIDDEN), gamma.reshape(1, HIDDEN),
                    beta.reshape(1, HIDDEN))
    return out[0]


# speedup vs baseline: 18.0914x; 1.1080x over previous
import functools

import jax
import jax.numpy as jnp
from jax import lax
from jax.experimental import pallas as pl
from jax.experimental.pallas import tpu as pltpu
from jax.experimental.pallas import tpu_sc as plsc

N = 50000
E = 1600000
N_OPS = 128
HIDDEN = 512

NC = 2
NS = 16
NW = NC * NS

BLK = 512
GRID = 98
N_PAD = GRID * BLK
RPS = N_PAD // NS
ZROWS = 784

MB = 128
EPT = E // NW
NFULL = EPT // MB
TAIL = EPT - NFULL * MB
NPAIR = NFULL // 2
RPB = N_PAD // NW


def _build_body(xnum, opidx, emb, xs_out, xnb, oib, embv, xb):
    c = lax.axis_index("c")
    s = lax.axis_index("s")
    wid = s * NC + c
    r0 = pl.multiple_of(wid * RPB, 8)
    pltpu.sync_copy(xnum.at[pl.ds(r0 * 2, RPB * 2)], xnb)
    pltpu.sync_copy(opidx.at[pl.ds(r0, RPB)], oib)
    pltpu.sync_copy(emb, embv)

    iota16 = lax.broadcasted_iota(jnp.int32, (16,), 0)
    z16 = jnp.zeros((16,), jnp.float32)

    def _bld(g, carry):
        rows16 = g * 16 + iota16
        idx16 = oib[pl.ds(g * 16, 16)]
        for col in range(2):
            cc = jnp.full((16,), col, jnp.int32)
            v = plsc.load_gather(xnb, [rows16 * 2 + col])
            plsc.store_scatter(xb, [rows16, cc], v)
        for ec in range(8):
            v = plsc.load_gather(embv, [idx16, jnp.full((16,), ec, jnp.int32)])
            plsc.store_scatter(
                xb, [rows16, jnp.full((16,), 2 + ec, jnp.int32)], v)
        for zc in range(10, 16):
            plsc.store_scatter(
                xb, [rows16, jnp.full((16,), zc, jnp.int32)], z16)
        return carry
    lax.fori_loop(0, RPB // 16, _bld, 0)
    pltpu.sync_copy(xb, xs_out.at[pl.ds(r0, RPB)])


def _build_x(xnum_p, opidx_p, emb):
    mesh = plsc.VectorSubcoreMesh(core_axis_name="c", subcore_axis_name="s")
    fn = functools.partial(
        pl.kernel,
        out_type=jax.ShapeDtypeStruct((N_PAD, 16), jnp.float32),
        mesh=mesh,
        compiler_params=pltpu.CompilerParams(
            use_tc_tiling_on_sc=False, needs_layout_passes=False),
        scratch_types=[
            pltpu.VMEM((RPB * 2,), jnp.float32),
            pltpu.VMEM((RPB,), jnp.int32),
            pltpu.VMEM((N_OPS, 8), jnp.float32),
            pltpu.VMEM((RPB, 16), jnp.float32),
        ],
    )(_build_body)
    return fn(xnum_p, opidx_p, emb)


def _edge_body(xpad, edges, agg_hbm,
               s_a, s_b, d_a, d_b, r_a, r_b, s_t, d_t, r_t, zbuf, aggs,
               gs_a, gs_b, is_a, is_b, ss_a, ss_b):
    c = lax.axis_index("c")
    s = lax.axis_index("s")
    wid = s * NC + c

    sbase = pl.multiple_of(s * RPS, 8)

    @pl.when(c == 0)
    def _():
        pltpu.sync_copy(xpad.at[pl.ds(sbase, RPS)],
                        aggs.at[pl.ds(sbase, RPS)])

    @pl.when(c != 0)
    def _():
        z16 = jnp.zeros((16,), jnp.float32)

        def _zrow(i, carry):
            zbuf[i, :] = z16
            return carry
        lax.fori_loop(0, ZROWS, _zrow, 0)
        for q in range(RPS // ZROWS):
            pltpu.sync_copy(zbuf, aggs.at[pl.ds(sbase + q * ZROWS, ZROWS)])
    plsc.subcore_barrier()

    e0 = pl.multiple_of(wid * EPT, 8)

    def idx_copy(m, sb, db, isem):
        off = pl.multiple_of(e0 + m * MB, 8)
        cp1 = pltpu.make_async_copy(edges.at[pl.ds(off, MB)], sb, isem)
        cp1.start()
        cp2 = pltpu.make_async_copy(edges.at[pl.ds(E + off, MB)], db, isem)
        cp2.start()
        return cp1, cp2

    pltpu.sync_copy(edges.at[pl.ds(e0, MB)], s_a)
    pltpu.sync_copy(edges.at[pl.ds(E + e0, MB)], d_a)
    pltpu.make_async_copy(xpad.at[s_a], r_a, gs_a).start()

    def _pair(p, carry):
        m0 = 2 * p
        cps, cpd = idx_copy(m0 + 1, s_b, d_b, is_b)
        pltpu.make_async_copy(xpad.at[s_a], r_a, gs_a).wait()
        sc_a = pltpu.async_copy(r_a, aggs.at[d_a], ss_a, add=True)
        cps.wait()
        cpd.wait()
        pltpu.make_async_copy(xpad.at[s_b], r_b, gs_b).start()
        sc_a.wait()
        cps, cpd = idx_copy(m0 + 2, s_a, d_a, is_a)
        pltpu.make_async_copy(xpad.at[s_b], r_b, gs_b).wait()
        sc_b = pltpu.async_copy(r_b, aggs.at[d_b], ss_b, add=True)
        cps.wait()
        cpd.wait()
        pltpu.make_async_copy(xpad.at[s_a], r_a, gs_a).start()
        sc_b.wait()
        return carry
    lax.fori_loop(0, NPAIR - 1, _pair, 0)

    cps, cpd = idx_copy(NFULL - 1, s_b, d_b, is_b)
    pltpu.make_async_copy(xpad.at[s_a], r_a, gs_a).wait()
    pltpu.sync_copy(r_a, aggs.at[d_a], add=True)
    cps.wait()
    cpd.wait()
    pltpu.make_async_copy(xpad.at[s_b], r_b, gs_b).start()
    pltpu.make_async_copy(xpad.at[s_b], r_b, gs_b).wait()
    pltpu.sync_copy(r_b, aggs.at[d_b], add=True)

    off_t = pl.multiple_of(e0 + NFULL * MB, 8)
    pltpu.sync_copy(edges.at[pl.ds(off_t, TAIL)], s_t)
    pltpu.sync_copy(edges.at[pl.ds(E + off_t, TAIL)], d_t)
    cp_t = pltpu.make_async_copy(xpad.at[s_t], r_t, gs_a)
    cp_t.start()
    cp_t.wait()
    pltpu.sync_copy(r_t, aggs.at[d_t], add=True)

    plsc.subcore_barrier()
    pltpu.sync_copy(aggs.at[pl.ds(sbase, RPS)],
                    agg_hbm.at[c, pl.ds(sbase, RPS)])


def _edge_agg(x_pad, edges):
    mesh = plsc.VectorSubcoreMesh(core_axis_name="c", subcore_axis_name="s")
    fn = functools.partial(
        pl.kernel,
        out_type=jax.ShapeDtypeStruct((NC, N_PAD, 16), jnp.float32),
        mesh=mesh,
        compiler_params=pltpu.CompilerParams(use_tc_tiling_on_sc=False),
        scratch_types=[
            pltpu.VMEM((MB,), jnp.int32),
            pltpu.VMEM((MB,), jnp.int32),
            pltpu.VMEM((MB,), jnp.int32),
            pltpu.VMEM((MB,), jnp.int32),
            pltpu.VMEM((MB, 16), jnp.float32),
            pltpu.VMEM((MB, 16), jnp.float32),
            pltpu.VMEM((TAIL,), jnp.int32),
            pltpu.VMEM((TAIL,), jnp.int32),
            pltpu.VMEM((TAIL, 16), jnp.float32),
            pltpu.VMEM((ZROWS, 16), jnp.float32),
            pltpu.VMEM_SHARED((N_PAD, 16), jnp.float32),
            pltpu.SemaphoreType.DMA,
            pltpu.SemaphoreType.DMA,
            pltpu.SemaphoreType.DMA,
            pltpu.SemaphoreType.DMA,
            pltpu.SemaphoreType.DMA,
            pltpu.SemaphoreType.DMA,
        ],
    )(_edge_body)
    return fn(x_pad, edges)


def _mlp_body(a_ref, w1_ref, b1_ref, w2_ref, b2_ref,
              g_ref, be_ref, o_ref):
    m = pl.program_id(0)
    h0 = a_ref[0] + a_ref[1]
    h1 = jnp.maximum(
        jnp.dot(h0, w1_ref[...], preferred_element_type=jnp.float32)
        + b1_ref[...], 0.0)
    h2 = jnp.dot(h1.astype(jnp.bfloat16), w2_ref[...],
                 preferred_element_type=jnp.float32) + b2_ref[...]
    mu = jnp.mean(h2, axis=-1, keepdims=True)
    d = h2 - mu
    var = jnp.mean(d * d, axis=-1, keepdims=True)
    hn = d * lax.rsqrt(var + 1e-5)

    @pl.when(m == 0)
    def _():
        o_ref[...] = jnp.zeros_like(o_ref)

    @pl.when(m < GRID - 1)
    def _():
        o_ref[...] += jnp.sum(hn, axis=0, keepdims=True)

    @pl.when(m == GRID - 1)
    def _():
        rows = m * BLK + lax.broadcasted_iota(jnp.int32, (BLK, 1), 0)
        hnm = jnp.where(rows < N, hn, 0.0)
        acc = o_ref[...] + jnp.sum(hnm, axis=0, keepdims=True)
        o_ref[...] = acc * (g_ref[...] * (1.0 / N)) + be_ref[...]


def _mlp_mean(agg, w1p, b1, w2, b2, gamma, beta):
    row512 = pl.BlockSpec((1, HIDDEN), lambda m: (0, 0))
    return pl.pallas_call(
        _mlp_body,
        grid=(GRID,),
        in_specs=[
            pl.BlockSpec((NC, BLK, 16), lambda m: (0, m, 0)),
            pl.BlockSpec((16, HIDDEN), lambda m: (0, 0)),
            row512,
            pl.BlockSpec((HIDDEN, HIDDEN), lambda m: (0, 0)),
            row512, row512, row512,
        ],
        out_specs=pl.BlockSpec((1, HIDDEN), lambda m: (0, 0)),
        out_shape=jax.ShapeDtypeStruct((1, HIDDEN), jnp.float32),
    )(agg, w1p, b1, w2, b2, gamma, beta)


@jax.jit
def kernel(x_num, op_idx, edge_index, op_emb, W1, b1, W2, b2, gamma, beta):
    op_idx = op_idx.astype(jnp.int32)
    edge_index = edge_index.astype(jnp.int32)

    xnum_f = jnp.zeros((N_PAD * 2,), jnp.float32).at[:2 * N].set(
        x_num.reshape(2 * N))
    opidx_p = jnp.zeros((N_PAD,), jnp.int32).at[:N].set(op_idx)
    w1p = jnp.zeros((16, HIDDEN), jnp.float32).at[0:10, :].set(W1)

    x_pad = _build_x(xnum_f, opidx_p, op_emb)
    agg = _edge_agg(x_pad, edge_index.reshape(2 * E))
    out = _mlp_mean(agg,
                    w1p, b1.reshape(1, HIDDEN), W2.astype(jnp.bfloat16),
                    b2.reshape(1, HIDDEN), gamma.reshape(1, HIDDEN),
                    beta.reshape(1, HIDDEN))
    return out[0]


# speedup vs baseline: 27.8833x; 1.5412x over previous
import functools

import jax
import jax.numpy as jnp
from jax import lax
from jax.experimental import pallas as pl
from jax.experimental.pallas import tpu as pltpu
from jax.experimental.pallas import tpu_sc as plsc

N = 50000
E = 1600000
N_OPS = 128
HIDDEN = 512

NC = 2
NS = 16
NW = NC * NS

BLK = 512
GRID = 98
N_PAD = GRID * BLK
RPS = N_PAD // NS
ZROWS = 784

MB = 128
EPT = E // NW
NFULL = EPT // MB
TAIL = EPT - NFULL * MB
NSET = 6
NGRP = NFULL // NSET
RPB = N_PAD // NW


def _build_body(xnum, opidx, emb, xs_out, xnb, oib, embv, xb):
    c = lax.axis_index("c")
    s = lax.axis_index("s")
    wid = s * NC + c
    r0 = pl.multiple_of(wid * RPB, 8)
    pltpu.sync_copy(xnum.at[pl.ds(r0 * 2, RPB * 2)], xnb)
    pltpu.sync_copy(opidx.at[pl.ds(r0, RPB)], oib)
    pltpu.sync_copy(emb, embv)

    iota16 = lax.broadcasted_iota(jnp.int32, (16,), 0)
    z16 = jnp.zeros((16,), jnp.float32)

    def _bld(g, carry):
        rows16 = g * 16 + iota16
        idx16 = oib[pl.ds(g * 16, 16)]
        for col in range(2):
            cc = jnp.full((16,), col, jnp.int32)
            v = plsc.load_gather(xnb, [rows16 * 2 + col])
            plsc.store_scatter(xb, [rows16, cc], v)
        for ec in range(8):
            v = plsc.load_gather(embv, [idx16, jnp.full((16,), ec, jnp.int32)])
            plsc.store_scatter(
                xb, [rows16, jnp.full((16,), 2 + ec, jnp.int32)], v)
        for zc in range(10, 16):
            plsc.store_scatter(
                xb, [rows16, jnp.full((16,), zc, jnp.int32)], z16)
        return carry
    lax.fori_loop(0, RPB // 16, _bld, 0)
    pltpu.sync_copy(xb, xs_out.at[pl.ds(r0, RPB)])


def _build_x(xnum_p, opidx_p, emb):
    mesh = plsc.VectorSubcoreMesh(core_axis_name="c", subcore_axis_name="s")
    fn = functools.partial(
        pl.kernel,
        out_type=jax.ShapeDtypeStruct((N_PAD, 16), jnp.float32),
        mesh=mesh,
        compiler_params=pltpu.CompilerParams(
            use_tc_tiling_on_sc=False, needs_layout_passes=False),
        scratch_types=[
            pltpu.VMEM((RPB * 2,), jnp.float32),
            pltpu.VMEM((RPB,), jnp.int32),
            pltpu.VMEM((N_OPS, 8), jnp.float32),
            pltpu.VMEM((RPB, 16), jnp.float32),
        ],
    )(_build_body)
    return fn(xnum_p, opidx_p, emb)


def _edge_body(xpad, edges, agg_hbm,
               sbuf, dbuf, rbuf, s_t, d_t, r_t, zbuf, aggs,
               gsem, isem, ssem):
    c = lax.axis_index("c")
    s = lax.axis_index("s")
    wid = s * NC + c

    sbase = pl.multiple_of(s * RPS, 8)

    @pl.when(c == 0)
    def _():
        pltpu.sync_copy(xpad.at[pl.ds(sbase, RPS)],
                        aggs.at[pl.ds(sbase, RPS)])

    @pl.when(c != 0)
    def _():
        z16 = jnp.zeros((16,), jnp.float32)

        def _zrow(i, carry):
            zbuf[i, :] = z16
            return carry
        lax.fori_loop(0, ZROWS, _zrow, 0)
        for q in range(RPS // ZROWS):
            pltpu.sync_copy(zbuf, aggs.at[pl.ds(sbase + q * ZROWS, ZROWS)])
    plsc.subcore_barrier()

    e0 = pl.multiple_of(wid * EPT, 8)

    def idx_copy(m, u):
        off = pl.multiple_of(e0 + m * MB, 8)
        cp1 = pltpu.make_async_copy(
            edges.at[pl.ds(off, MB)], sbuf[u], isem[u])
        cp1.start()
        cp2 = pltpu.make_async_copy(
            edges.at[pl.ds(E + off, MB)], dbuf[u], isem[u])
        cp2.start()
        return cp1, cp2

    def gather(u):
        return pltpu.make_async_copy(xpad.at[sbuf[u]], rbuf[u], gsem[u])

    for u in range(NSET):
        cp1, cp2 = idx_copy(u, u)
        cp1.wait()
        cp2.wait()
    for u in range(NSET):
        gather(u).start()

    def _group(g, carry):
        m0 = g * NSET
        scs = []
        for u in range(NSET):
            gather(u).wait()
            scs.append(
                pltpu.async_copy(rbuf[u], aggs.at[dbuf[u]], ssem[u],
                                 add=True))
        idxs = []
        for u in range(NSET):
            scs[u].wait()
            idxs.append(idx_copy(m0 + NSET + u, u))
        for u in range(NSET):
            idxs[u][0].wait()
            idxs[u][1].wait()
            gather(u).start()
        return carry
    lax.fori_loop(0, NGRP - 1, _group, 0)

    scs = []
    for u in range(NSET):
        gather(u).wait()
        scs.append(
            pltpu.async_copy(rbuf[u], aggs.at[dbuf[u]], ssem[u], add=True))
    for u in range(NSET):
        scs[u].wait()

    off_t = pl.multiple_of(e0 + NFULL * MB, 8)
    pltpu.sync_copy(edges.at[pl.ds(off_t, TAIL)], s_t)
    pltpu.sync_copy(edges.at[pl.ds(E + off_t, TAIL)], d_t)
    cp_t = pltpu.make_async_copy(xpad.at[s_t], r_t, gsem[0])
    cp_t.start()
    cp_t.wait()
    pltpu.sync_copy(r_t, aggs.at[d_t], add=True)

    plsc.subcore_barrier()
    pltpu.sync_copy(aggs.at[pl.ds(sbase, RPS)],
                    agg_hbm.at[c, pl.ds(sbase, RPS)])


def _edge_agg(x_pad, edges):
    mesh = plsc.VectorSubcoreMesh(core_axis_name="c", subcore_axis_name="s")
    fn = functools.partial(
        pl.kernel,
        out_type=jax.ShapeDtypeStruct((NC, N_PAD, 16), jnp.float32),
        mesh=mesh,
        compiler_params=pltpu.CompilerParams(use_tc_tiling_on_sc=False),
        scratch_types=[
            [pltpu.VMEM((MB,), jnp.int32)] * NSET,
            [pltpu.VMEM((MB,), jnp.int32)] * NSET,
            [pltpu.VMEM((MB, 16), jnp.float32)] * NSET,
            pltpu.VMEM((TAIL,), jnp.int32),
            pltpu.VMEM((TAIL,), jnp.int32),
            pltpu.VMEM((TAIL, 16), jnp.float32),
            pltpu.VMEM((ZROWS, 16), jnp.float32),
            pltpu.VMEM_SHARED((N_PAD, 16), jnp.float32),
            [pltpu.SemaphoreType.DMA] * NSET,
            [pltpu.SemaphoreType.DMA] * NSET,
            [pltpu.SemaphoreType.DMA] * NSET,
        ],
    )(_edge_body)
    return fn(x_pad, edges)


def _mlp_body(a_ref, w1_ref, b1_ref, w2_ref, b2_ref,
              g_ref, be_ref, o_ref):
    m = pl.program_id(0)
    h0 = a_ref[0] + a_ref[1]
    h1 = jnp.maximum(
        jnp.dot(h0, w1_ref[...], preferred_element_type=jnp.float32)
        + b1_ref[...], 0.0)
    h2 = jnp.dot(h1.astype(jnp.bfloat16), w2_ref[...],
                 preferred_element_type=jnp.float32) + b2_ref[...]
    mu = jnp.mean(h2, axis=-1, keepdims=True)
    d = h2 - mu
    var = jnp.mean(d * d, axis=-1, keepdims=True)
    hn = d * lax.rsqrt(var + 1e-5)

    @pl.when(m == 0)
    def _():
        o_ref[...] = jnp.zeros_like(o_ref)

    @pl.when(m < GRID - 1)
    def _():
        o_ref[...] += jnp.sum(hn, axis=0, keepdims=True)

    @pl.when(m == GRID - 1)
    def _():
        rows = m * BLK + lax.broadcasted_iota(jnp.int32, (BLK, 1), 0)
        hnm = jnp.where(rows < N, hn, 0.0)
        acc = o_ref[...] + jnp.sum(hnm, axis=0, keepdims=True)
        o_ref[...] = acc * (g_ref[...] * (1.0 / N)) + be_ref[...]


def _mlp_mean(agg, w1p, b1, w2, b2, gamma, beta):
    row512 = pl.BlockSpec((1, HIDDEN), lambda m: (0, 0))
    return pl.pallas_call(
        _mlp_body,
        grid=(GRID,),
        in_specs=[
            pl.BlockSpec((NC, BLK, 16), lambda m: (0, m, 0)),
            pl.BlockSpec((16, HIDDEN), lambda m: (0, 0)),
            row512,
            pl.BlockSpec((HIDDEN, HIDDEN), lambda m: (0, 0)),
            row512, row512, row512,
        ],
        out_specs=pl.BlockSpec((1, HIDDEN), lambda m: (0, 0)),
        out_shape=jax.ShapeDtypeStruct((1, HIDDEN), jnp.float32),
    )(agg, w1p, b1, w2, b2, gamma, beta)


@jax.jit
def kernel(x_num, op_idx, edge_index, op_emb, W1, b1, W2, b2, gamma, beta):
    op_idx = op_idx.astype(jnp.int32)
    edge_index = edge_index.astype(jnp.int32)

    xnum_f = jnp.zeros((N_PAD * 2,), jnp.float32).at[:2 * N].set(
        x_num.reshape(2 * N))
    opidx_p = jnp.zeros((N_PAD,), jnp.int32).at[:N].set(op_idx)
    w1p = jnp.zeros((16, HIDDEN), jnp.float32).at[0:10, :].set(W1)

    x_pad = _build_x(xnum_f, opidx_p, op_emb)
    agg = _edge_agg(x_pad, edge_index.reshape(2 * E))
    out = _mlp_mean(agg,
                    w1p, b1.reshape(1, HIDDEN), W2.astype(jnp.bfloat16),
                    b2.reshape(1, HIDDEN), gamma.reshape(1, HIDDEN),
                    beta.reshape(1, HIDDEN))
    return out[0]


# speedup vs baseline: 29.4554x; 1.0564x over previous
import functools

import jax
import jax.numpy as jnp
from jax import lax
from jax.experimental import pallas as pl
from jax.experimental.pallas import tpu as pltpu
from jax.experimental.pallas import tpu_sc as plsc

N = 50000
E = 1600000
N_OPS = 128
HIDDEN = 512

NC = 2
NS = 16
NW = NC * NS

BLK = 512
GRID = 98
N_PAD = GRID * BLK
RPS = N_PAD // NS
ZROWS = 784

MB = 128
EPT = E // NW
NFULL = EPT // MB
TAIL = EPT - NFULL * MB
NSET = 10
NGRP = NFULL // NSET
RPB = N_PAD // NW


def _build_body(xnum, opidx, emb, xs_out, xnb, oib, embv, xb):
    c = lax.axis_index("c")
    s = lax.axis_index("s")
    wid = s * NC + c
    r0 = pl.multiple_of(wid * RPB, 8)
    pltpu.sync_copy(xnum.at[pl.ds(r0 * 2, RPB * 2)], xnb)
    pltpu.sync_copy(opidx.at[pl.ds(r0, RPB)], oib)
    pltpu.sync_copy(emb, embv)

    iota16 = lax.broadcasted_iota(jnp.int32, (16,), 0)
    z16 = jnp.zeros((16,), jnp.float32)

    def _bld(g, carry):
        rows16 = g * 16 + iota16
        idx16 = oib[pl.ds(g * 16, 16)]
        for col in range(2):
            cc = jnp.full((16,), col, jnp.int32)
            v = plsc.load_gather(xnb, [rows16 * 2 + col])
            plsc.store_scatter(xb, [rows16, cc], v)
        for ec in range(8):
            v = plsc.load_gather(embv, [idx16, jnp.full((16,), ec, jnp.int32)])
            plsc.store_scatter(
                xb, [rows16, jnp.full((16,), 2 + ec, jnp.int32)], v)
        for zc in range(10, 16):
            plsc.store_scatter(
                xb, [rows16, jnp.full((16,), zc, jnp.int32)], z16)
        return carry
    lax.fori_loop(0, RPB // 16, _bld, 0)
    pltpu.sync_copy(xb, xs_out.at[pl.ds(r0, RPB)])


def _build_x(xnum_p, opidx_p, emb):
    mesh = plsc.VectorSubcoreMesh(core_axis_name="c", subcore_axis_name="s")
    fn = functools.partial(
        pl.kernel,
        out_type=jax.ShapeDtypeStruct((N_PAD, 16), jnp.float32),
        mesh=mesh,
        compiler_params=pltpu.CompilerParams(
            use_tc_tiling_on_sc=False, needs_layout_passes=False),
        scratch_types=[
            pltpu.VMEM((RPB * 2,), jnp.float32),
            pltpu.VMEM((RPB,), jnp.int32),
            pltpu.VMEM((N_OPS, 8), jnp.float32),
            pltpu.VMEM((RPB, 16), jnp.float32),
        ],
    )(_build_body)
    return fn(xnum_p, opidx_p, emb)


def _edge_body(xpad, edges, agg_hbm,
               sbuf, dbuf, rbuf, s_t, d_t, r_t, zbuf, aggs,
               gsem, isem, ssem):
    c = lax.axis_index("c")
    s = lax.axis_index("s")
    wid = s * NC + c

    sbase = pl.multiple_of(s * RPS, 8)

    @pl.when(c == 0)
    def _():
        pltpu.sync_copy(xpad.at[pl.ds(sbase, RPS)],
                        aggs.at[pl.ds(sbase, RPS)])

    @pl.when(c != 0)
    def _():
        z16 = jnp.zeros((16,), jnp.float32)

        def _zrow(i, carry):
            zbuf[i, :] = z16
            return carry
        lax.fori_loop(0, ZROWS, _zrow, 0)
        for q in range(RPS // ZROWS):
            pltpu.sync_copy(zbuf, aggs.at[pl.ds(sbase + q * ZROWS, ZROWS)])
    plsc.subcore_barrier()

    e0 = pl.multiple_of(wid * EPT, 8)

    def idx_copy(m, u):
        off = pl.multiple_of(e0 + m * MB, 8)
        cp1 = pltpu.make_async_copy(
            edges.at[pl.ds(off, MB)], sbuf[u], isem[u])
        cp1.start()
        cp2 = pltpu.make_async_copy(
            edges.at[pl.ds(E + off, MB)], dbuf[u], isem[u])
        cp2.start()
        return cp1, cp2

    def gather(u):
        return pltpu.make_async_copy(xpad.at[sbuf[u]], rbuf[u], gsem[u])

    for u in range(NSET):
        cp1, cp2 = idx_copy(u, u)
        cp1.wait()
        cp2.wait()
    for u in range(NSET):
        gather(u).start()

    def _group(g, carry):
        m0 = g * NSET
        scs = []
        for u in range(NSET):
            gather(u).wait()
            scs.append(
                pltpu.async_copy(rbuf[u], aggs.at[dbuf[u]], ssem[u],
                                 add=True))
        idxs = []
        for u in range(NSET):
            scs[u].wait()
            idxs.append(idx_copy(m0 + NSET + u, u))
        for u in range(NSET):
            idxs[u][0].wait()
            idxs[u][1].wait()
            gather(u).start()
        return carry
    lax.fori_loop(0, NGRP - 1, _group, 0)

    scs = []
    for u in range(NSET):
        gather(u).wait()
        scs.append(
            pltpu.async_copy(rbuf[u], aggs.at[dbuf[u]], ssem[u], add=True))
    for u in range(NSET):
        scs[u].wait()

    off_t = pl.multiple_of(e0 + NFULL * MB, 8)
    pltpu.sync_copy(edges.at[pl.ds(off_t, TAIL)], s_t)
    pltpu.sync_copy(edges.at[pl.ds(E + off_t, TAIL)], d_t)
    cp_t = pltpu.make_async_copy(xpad.at[s_t], r_t, gsem[0])
    cp_t.start()
    cp_t.wait()
    pltpu.sync_copy(r_t, aggs.at[d_t], add=True)

    plsc.subcore_barrier()
    pltpu.sync_copy(aggs.at[pl.ds(sbase, RPS)],
                    agg_hbm.at[c, pl.ds(sbase, RPS)])


def _edge_agg(x_pad, edges):
    mesh = plsc.VectorSubcoreMesh(core_axis_name="c", subcore_axis_name="s")
    fn = functools.partial(
        pl.kernel,
        out_type=jax.ShapeDtypeStruct((NC, N_PAD, 16), jnp.float32),
        mesh=mesh,
        compiler_params=pltpu.CompilerParams(use_tc_tiling_on_sc=False),
        scratch_types=[
            [pltpu.VMEM((MB,), jnp.int32)] * NSET,
            [pltpu.VMEM((MB,), jnp.int32)] * NSET,
            [pltpu.VMEM((MB, 16), jnp.float32)] * NSET,
            pltpu.VMEM((TAIL,), jnp.int32),
            pltpu.VMEM((TAIL,), jnp.int32),
            pltpu.VMEM((TAIL, 16), jnp.float32),
            pltpu.VMEM((ZROWS, 16), jnp.float32),
            pltpu.VMEM_SHARED((N_PAD, 16), jnp.float32),
            [pltpu.SemaphoreType.DMA] * NSET,
            [pltpu.SemaphoreType.DMA] * NSET,
            [pltpu.SemaphoreType.DMA] * NSET,
        ],
    )(_edge_body)
    return fn(x_pad, edges)


def _mlp_body(a_ref, w1_ref, b1_ref, w2_ref, b2_ref,
              g_ref, be_ref, o_ref):
    m = pl.program_id(0)
    h0 = a_ref[0] + a_ref[1]
    h1 = jnp.maximum(
        jnp.dot(h0, w1_ref[...], preferred_element_type=jnp.float32)
        + b1_ref[...], 0.0)
    h2 = jnp.dot(h1.astype(jnp.bfloat16), w2_ref[...],
                 preferred_element_type=jnp.float32) + b2_ref[...]
    mu = jnp.mean(h2, axis=-1, keepdims=True)
    d = h2 - mu
    var = jnp.mean(d * d, axis=-1, keepdims=True)
    hn = d * lax.rsqrt(var + 1e-5)

    @pl.when(m == 0)
    def _():
        o_ref[...] = jnp.zeros_like(o_ref)

    @pl.when(m < GRID - 1)
    def _():
        o_ref[...] += jnp.sum(hn, axis=0, keepdims=True)

    @pl.when(m == GRID - 1)
    def _():
        rows = m * BLK + lax.broadcasted_iota(jnp.int32, (BLK, 1), 0)
        hnm = jnp.where(rows < N, hn, 0.0)
        acc = o_ref[...] + jnp.sum(hnm, axis=0, keepdims=True)
        o_ref[...] = acc * (g_ref[...] * (1.0 / N)) + be_ref[...]


def _mlp_mean(agg, w1p, b1, w2, b2, gamma, beta):
    row512 = pl.BlockSpec((1, HIDDEN), lambda m: (0, 0))
    return pl.pallas_call(
        _mlp_body,
        grid=(GRID,),
        in_specs=[
            pl.BlockSpec((NC, BLK, 16), lambda m: (0, m, 0)),
            pl.BlockSpec((16, HIDDEN), lambda m: (0, 0)),
            row512,
            pl.BlockSpec((HIDDEN, HIDDEN), lambda m: (0, 0)),
            row512, row512, row512,
        ],
        out_specs=pl.BlockSpec((1, HIDDEN), lambda m: (0, 0)),
        out_shape=jax.ShapeDtypeStruct((1, HIDDEN), jnp.float32),
    )(agg, w1p, b1, w2, b2, gamma, beta)


@jax.jit
def kernel(x_num, op_idx, edge_index, op_emb, W1, b1, W2, b2, gamma, beta):
    op_idx = op_idx.astype(jnp.int32)
    edge_index = edge_index.astype(jnp.int32)

    xnum_f = jnp.zeros((N_PAD * 2,), jnp.float32).at[:2 * N].set(
        x_num.reshape(2 * N))
    opidx_p = jnp.zeros((N_PAD,), jnp.int32).at[:N].set(op_idx)
    w1p = jnp.zeros((16, HIDDEN), jnp.float32).at[0:10, :].set(W1)

    x_pad = _build_x(xnum_f, opidx_p, op_emb)
    agg = _edge_agg(x_pad, edge_index.reshape(2 * E))
    out = _mlp_mean(agg,
                    w1p, b1.reshape(1, HIDDEN), W2.astype(jnp.bfloat16),
                    b2.reshape(1, HIDDEN), gamma.reshape(1, HIDDEN),
                    beta.reshape(1, HIDDEN))
    return out[0]
